# baseline (speedup 1.0000x reference)
import functools

import jax
import jax.numpy as jnp
from jax import lax
from jax.experimental import pallas as pl
from jax.experimental.pallas import tpu as pltpu
from jax.experimental.pallas import tpu_sc as plsc

_T = 8192
_D = 2048
_E = 8
_BT = 512
_BR = 256
_S = _T * 2 + _E * _BR
_NBLK = _S // _BR
_LANES = 128
_NEG = -1e30
_NW = 32
_TPW = _T // _NW
_CS = 64
_CG = 32


def _bf16_bits(v):
    ai = lax.bitcast_convert_type(v, jnp.int32)
    rb = lax.shift_right_logical(ai, 16) & 1
    return lax.shift_right_logical(ai + 0x7FFF + rb, 16) & 0xFFFF


def _pack_halves(v):
    h = v.shape[1] // 2
    lo = _bf16_bits(v[:, :h])
    hi = _bf16_bits(v[:, h:])
    return lo | lax.shift_left(hi, 16)


def _unpack_halves(pk):
    lo = lax.bitcast_convert_type(lax.shift_left(pk, 16), jnp.float32)
    hi = lax.bitcast_convert_type(pk & jnp.int32(-65536), jnp.float32)
    return jnp.concatenate([lo, hi], axis=1)


def _route_body(x_ref, gw_ref, gb_ref, sc_ref, wts_ref, em_ref, rk_ref,
                cnt_ref, xpk_ref, carry_ref):
    i = pl.program_id(0)

    @pl.when(i == 0)
    def _():
        carry_ref[...] = jnp.zeros_like(carry_ref)

    xv = x_ref[...]
    xpk_ref[...] = _pack_halves(xv)
    logits = jnp.dot(xv, gw_ref[...],
                     preferred_element_type=jnp.float32) + gb_ref[...]
    m = jnp.max(logits, axis=1, keepdims=True)
    p = jnp.exp(logits - m)
    scores = p / jnp.sum(p, axis=1, keepdims=True)
    sc_ref[...] = scores
    lane = lax.broadcasted_iota(jnp.int32, scores.shape, 1)
    m1 = jnp.max(scores, axis=1, keepdims=True)
    e0 = jnp.min(jnp.where(scores == m1, lane, _LANES), axis=1, keepdims=True)
    sc2 = jnp.where(lane == e0, -1.0, scores)
    m2 = jnp.max(sc2, axis=1, keepdims=True)
    e1 = jnp.min(jnp.where(sc2 == m2, lane, _LANES), axis=1, keepdims=True)
    wts_ref[...] = jnp.where(lane == 0, m1, jnp.where(lane == 1, m2, 0.0))
    em_ref[...] = jnp.where(lane == 0, e0, jnp.where(lane == 1, e1, 0))

    oh0 = (lane == e0).astype(jnp.float32)
    oh1 = (lane == e1).astype(jnp.float32)
    hist = oh0 + oh1
    r_i = lax.broadcasted_iota(jnp.int32, (_BT, _BT), 0)
    c_i = lax.broadcasted_iota(jnp.int32, (_BT, _BT), 1)
    tril = (c_i < r_i).astype(jnp.float32)
    excl = jnp.dot(tril, hist, preferred_element_type=jnp.float32)
    excl = excl + carry_ref[...]
    rank0 = jnp.sum(oh0 * excl, axis=1, keepdims=True)
    rank1 = jnp.sum(oh1 * excl, axis=1, keepdims=True)
    rk_ref[...] = jnp.where(lane == 0, rank0,
                            jnp.where(lane == 1, rank1, 0.0)).astype(jnp.int32)
    carry_ref[...] = carry_ref[...] + jnp.sum(hist, axis=0, keepdims=True)

    @pl.when(i == _T // _BT - 1)
    def _():
        cnt_ref[...] = carry_ref[...].astype(jnp.int32)


def _pos_body(cnt_ref, em_ref, rk_ref, posm_ref, be_ref):
    cnt = cnt_ref[...].astype(jnp.float32)
    cb = jnp.ceil(cnt / _BR) * _BR
    r_i = lax.broadcasted_iota(jnp.int32, (_LANES, _LANES), 0)
    c_i = lax.broadcasted_iota(jnp.int32, (_LANES, _LANES), 1)
    strict = (r_i < c_i).astype(jnp.float32)
    ab = jnp.dot(cb, strict, preferred_element_type=jnp.float32)

    em = em_ref[...]
    rk = rk_ref[...]
    lane = lax.broadcasted_iota(jnp.int32, em.shape, 1)
    e0 = em[:, 0:1]
    e1 = em[:, 1:2]
    p0 = jnp.sum(jnp.where(lane == e0, ab, 0.0), axis=1,
                 keepdims=True).astype(jnp.int32) + rk[:, 0:1]
    p1 = jnp.sum(jnp.where(lane == e1, ab, 0.0), axis=1,
                 keepdims=True).astype(jnp.int32) + rk[:, 1:2]
    pm = jnp.where(lane == 0, p0, jnp.where(lane == 1, p1, 0))
    posm_ref[...] = jnp.transpose(pm)[0:8, :]

    bl = lax.broadcasted_iota(
        jnp.int32, (1, 2 * _LANES), 1).astype(jnp.float32) * _BR
    acc = jnp.full((1, 2 * _LANES), -1, jnp.int32)
    lane1 = lax.broadcasted_iota(jnp.int32, (1, _LANES), 1)
    for e in range(_E):
        ab_e = jnp.sum(jnp.where(lane1 == e, ab, 0.0))
        acc = acc + (bl >= ab_e).astype(jnp.int32)
    be_ref[...] = acc


def _group_mm_body(be_ref, x_ref, w_ref, o_ref):
    del be_ref
    xb = _unpack_halves(x_ref[...]).astype(jnp.bfloat16)
    acc = jnp.dot(xb, w_ref[0, :, :], preferred_element_type=jnp.float32)
    o_ref[...] = _pack_halves(acc)


def _combine_body(y0_ref, y1_ref, wts_ref, sc_ref, bp_ref, o_ref):
    wts = wts_ref[...]
    acc = jnp.dot(sc_ref[...], bp_ref[...], preferred_element_type=jnp.float32)
    acc = acc + _unpack_halves(y0_ref[...]) * wts[:, 0:1]
    acc = acc + _unpack_halves(y1_ref[...]) * wts[:, 1:2]
    o_ref[...] = acc


def _sc_scatter(x_hbm, pos0_hbm, pos1_hbm, xs_hbm, rows, idx0, idx1,
                sem0, sem1):
    wid = lax.axis_index("s") * 2 + lax.axis_index("c")
    base = wid * _TPW

    def body(j, carry):
        b = base + j * _CS
        pltpu.sync_copy(pos0_hbm.at[pl.ds(b, _CS)], idx0)
        pltpu.sync_copy(pos1_hbm.at[pl.ds(b, _CS)], idx1)
        pltpu.sync_copy(x_hbm.at[pl.ds(b, _CS)], rows)
        c0 = pltpu.async_copy(rows, xs_hbm.at[idx0], sem0)
        c1 = pltpu.async_copy(rows, xs_hbm.at[idx1], sem1)
        c0.wait()
        c1.wait()
        return carry

    lax.fori_loop(0, _TPW // _CS, body, 0)


def _sc_gather(ys_hbm, pos0_hbm, pos1_hbm, y_hbm, rows0, rows1, idx0, idx1,
               sem0, sem1):
    wid = lax.axis_index("s") * 2 + lax.axis_index("c")
    base = wid * _TPW

    def body(j, carry):
        b = base + j * _CG
        pltpu.sync_copy(pos0_hbm.at[pl.ds(b, _CG)], idx0)
        pltpu.sync_copy(pos1_hbm.at[pl.ds(b, _CG)], idx1)
        c0 = pltpu.async_copy(ys_hbm.at[idx0], rows0, sem0)
        c1 = pltpu.async_copy(ys_hbm.at[idx1], rows1, sem1)
        c0.wait()
        c1.wait()
        pltpu.sync_copy(rows0, y_hbm.at[pl.ds(b, _CG)])
        pltpu.sync_copy(rows1, y_hbm.at[pl.ds(_T + b, _CG)])
        return carry

    lax.fori_loop(0, _TPW // _CG, body, 0)


def _sc_mesh():
    return plsc.VectorSubcoreMesh(core_axis_name="c", subcore_axis_name="s")


def _dispatch_scatter(xpk, pos0, pos1):
    return pl.kernel(
        _sc_scatter,
        mesh=_sc_mesh(),
        out_type=jax.ShapeDtypeStruct((_S, _D // 2), jnp.int32),
        scratch_types=[
            pltpu.VMEM((_CS, _D // 2), jnp.int32),
            pltpu.VMEM((_CS,), jnp.int32),
            pltpu.VMEM((_CS,), jnp.int32),
            pltpu.SemaphoreType.DMA,
            pltpu.SemaphoreType.DMA,
        ],
    )(xpk, pos0, pos1)


def _gather_back(ys, pos0, pos1):
    return pl.kernel(
        _sc_gather,
        mesh=_sc_mesh(),
        out_type=jax.ShapeDtypeStruct((2 * _T, _D // 2), jnp.int32),
        scratch_types=[
            pltpu.VMEM((_CG, _D // 2), jnp.int32),
            pltpu.VMEM((_CG, _D // 2), jnp.int32),
            pltpu.VMEM((_CG,), jnp.int32),
            pltpu.VMEM((_CG,), jnp.int32),
            pltpu.SemaphoreType.DMA,
            pltpu.SemaphoreType.DMA,
        ],
    )(ys, pos0, pos1)


def kernel(x, gate_W, gate_b, expert_W, expert_b):
    n_tb = _T // _BT
    gw_pad = jnp.zeros((_D, _LANES), jnp.float32).at[:, :_E].set(gate_W)
    gb_pad = jnp.full((1, _LANES), _NEG, jnp.float32).at[0, :_E].set(gate_b)
    bp = jnp.zeros((_LANES, _D), jnp.float32).at[:_E].set(expert_b)
    w_bf = expert_W.astype(jnp.bfloat16)

    sc, wts, em, rk, cnt, xpk = pl.pallas_call(
        _route_body,
        grid=(n_tb,),
        in_specs=[
            pl.BlockSpec((_BT, _D), lambda i: (i, 0)),
            pl.BlockSpec((_D, _LANES), lambda i: (0, 0)),
            pl.BlockSpec((1, _LANES), lambda i: (0, 0)),
        ],
        out_specs=[
            pl.BlockSpec((_BT, _LANES), lambda i: (i, 0)),
            pl.BlockSpec((_BT, _LANES), lambda i: (i, 0)),
            pl.BlockSpec((_BT, _LANES), lambda i: (i, 0)),
            pl.BlockSpec((_BT, _LANES), lambda i: (i, 0)),
            pl.BlockSpec((1, _LANES), lambda i: (0, 0)),
            pl.BlockSpec((_BT, _D // 2), lambda i: (i, 0)),
        ],
        out_shape=[
            jax.ShapeDtypeStruct((_T, _LANES), jnp.float32),
            jax.ShapeDtypeStruct((_T, _LANES), jnp.float32),
            jax.ShapeDtypeStruct((_T, _LANES), jnp.int32),
            jax.ShapeDtypeStruct((_T, _LANES), jnp.int32),
            jax.ShapeDtypeStruct((1, _LANES), jnp.int32),
            jax.ShapeDtypeStruct((_T, _D // 2), jnp.int32),
        ],
        scratch_shapes=[pltpu.VMEM((1, _LANES), jnp.float32)],
    )(x, gw_pad, gb_pad)

    posm, be = pl.pallas_call(
        _pos_body,
        grid=(n_tb,),
        in_specs=[
            pl.BlockSpec((1, _LANES), lambda i: (0, 0)),
            pl.BlockSpec((_BT, _LANES), lambda i: (i, 0)),
            pl.BlockSpec((_BT, _LANES), lambda i: (i, 0)),
        ],
        out_specs=[
            pl.BlockSpec((8, _BT), lambda i: (0, i)),
            pl.BlockSpec((1, 2 * _LANES), lambda i: (0, 0)),
        ],
        out_shape=[
            jax.ShapeDtypeStruct((8, _T), jnp.int32),
            jax.ShapeDtypeStruct((1, 2 * _LANES), jnp.int32),
        ],
    )(cnt, em, rk)

    pos0 = posm[0]
    pos1 = posm[1]
    xs = _dispatch_scatter(xpk, pos0, pos1)
    grid_spec = pltpu.PrefetchScalarGridSpec(
        num_scalar_prefetch=1,
        grid=(_NBLK,),
        in_specs=[
            pl.BlockSpec((_BR, _D // 2), lambda nb, be_r: (nb, 0)),
            pl.BlockSpec((1, _D, _D), lambda nb, be_r: (be_r[0, nb], 0, 0)),
        ],
        out_specs=pl.BlockSpec((_BR, _D // 2), lambda nb, be_r: (nb, 0)),
    )
    ys = pl.pallas_call(
        _group_mm_body,
        grid_spec=grid_spec,
        out_shape=jax.ShapeDtypeStruct((_S, _D // 2), jnp.int32),
    )(be, xs, w_bf)

    y = _gather_back(ys, pos0, pos1)
    out = pl.pallas_call(
        _combine_body,
        grid=(n_tb,),
        in_specs=[
            pl.BlockSpec((_BT, _D // 2), lambda i: (i, 0)),
            pl.BlockSpec((_BT, _D // 2), lambda i: (i + n_tb, 0)),
            pl.BlockSpec((_BT, _LANES), lambda i: (i, 0)),
            pl.BlockSpec((_BT, _LANES), lambda i: (i, 0)),
            pl.BlockSpec((_LANES, _D), lambda i: (0, 0)),
        ],
        out_specs=pl.BlockSpec((_BT, _D), lambda i: (i, 0)),
        out_shape=jax.ShapeDtypeStruct((_T, _D), jnp.float32),
    )(y, y, wts, sc, bp)
    return out


# speedup vs baseline: 1.0195x; 1.0195x over previous
import functools

import jax
import jax.numpy as jnp
from jax import lax
from jax.experimental import pallas as pl
from jax.experimental.pallas import tpu as pltpu
from jax.experimental.pallas import tpu_sc as plsc

_T = 8192
_D = 2048
_E = 8
_BT = 512
_BR = 512
_S = _T * 2 + _E * _BR
_NBLK = _S // _BR
_LANES = 128
_NEG = -1e30
_NW = 32
_TPW = _T // _NW
_CS = 64
_CG = 32


def _bf16_bits(v):
    ai = lax.bitcast_convert_type(v, jnp.int32)
    rb = lax.shift_right_logical(ai, 16) & 1
    return lax.shift_right_logical(ai + 0x7FFF + rb, 16) & 0xFFFF


def _pack_halves(v):
    h = v.shape[1] // 2
    lo = _bf16_bits(v[:, :h])
    hi = _bf16_bits(v[:, h:])
    return lo | lax.shift_left(hi, 16)


def _unpack_halves(pk):
    lo = lax.bitcast_convert_type(lax.shift_left(pk, 16), jnp.float32)
    hi = lax.bitcast_convert_type(pk & jnp.int32(-65536), jnp.float32)
    return jnp.concatenate([lo, hi], axis=1)


def _route_body(x_ref, gw_ref, gb_ref, sc_ref, wts_ref, em_ref, rk_ref,
                cnt_ref, xpk_ref, carry_ref):
    i = pl.program_id(0)

    @pl.when(i == 0)
    def _():
        carry_ref[...] = jnp.zeros_like(carry_ref)

    xv = x_ref[...]
    xpk_ref[...] = _pack_halves(xv)
    logits = jnp.dot(xv, gw_ref[...],
                     preferred_element_type=jnp.float32) + gb_ref[...]
    m = jnp.max(logits, axis=1, keepdims=True)
    p = jnp.exp(logits - m)
    scores = p / jnp.sum(p, axis=1, keepdims=True)
    sc_ref[...] = scores
    lane = lax.broadcasted_iota(jnp.int32, scores.shape, 1)
    m1 = jnp.max(scores, axis=1, keepdims=True)
    e0 = jnp.min(jnp.where(scores == m1, lane, _LANES), axis=1, keepdims=True)
    sc2 = jnp.where(lane == e0, -1.0, scores)
    m2 = jnp.max(sc2, axis=1, keepdims=True)
    e1 = jnp.min(jnp.where(sc2 == m2, lane, _LANES), axis=1, keepdims=True)
    wts_ref[...] = jnp.where(lane == 0, m1, jnp.where(lane == 1, m2, 0.0))
    em_ref[...] = jnp.where(lane == 0, e0, jnp.where(lane == 1, e1, 0))

    oh0 = (lane == e0).astype(jnp.float32)
    oh1 = (lane == e1).astype(jnp.float32)
    hist = oh0 + oh1
    r_i = lax.broadcasted_iota(jnp.int32, (_BT, _BT), 0)
    c_i = lax.broadcasted_iota(jnp.int32, (_BT, _BT), 1)
    tril = (c_i < r_i).astype(jnp.float32)
    excl = jnp.dot(tril, hist, preferred_element_type=jnp.float32)
    excl = excl + carry_ref[...]
    rank0 = jnp.sum(oh0 * excl, axis=1, keepdims=True)
    rank1 = jnp.sum(oh1 * excl, axis=1, keepdims=True)
    rk_ref[...] = jnp.where(lane == 0, rank0,
                            jnp.where(lane == 1, rank1, 0.0)).astype(jnp.int32)
    carry_ref[...] = carry_ref[...] + jnp.sum(hist, axis=0, keepdims=True)

    @pl.when(i == _T // _BT - 1)
    def _():
        cnt_ref[...] = carry_ref[...].astype(jnp.int32)


def _pos_body(cnt_ref, em_ref, rk_ref, posm_ref, be_ref):
    cnt = cnt_ref[...].astype(jnp.float32)
    cb = jnp.ceil(cnt / _BR) * _BR
    r_i = lax.broadcasted_iota(jnp.int32, (_LANES, _LANES), 0)
    c_i = lax.broadcasted_iota(jnp.int32, (_LANES, _LANES), 1)
    strict = (r_i < c_i).astype(jnp.float32)
    ab = jnp.dot(cb, strict, preferred_element_type=jnp.float32)

    em = em_ref[...]
    rk = rk_ref[...]
    lane = lax.broadcasted_iota(jnp.int32, em.shape, 1)
    e0 = em[:, 0:1]
    e1 = em[:, 1:2]
    p0 = jnp.sum(jnp.where(lane == e0, ab, 0.0), axis=1,
                 keepdims=True).astype(jnp.int32) + rk[:, 0:1]
    p1 = jnp.sum(jnp.where(lane == e1, ab, 0.0), axis=1,
                 keepdims=True).astype(jnp.int32) + rk[:, 1:2]
    pm = jnp.where(lane == 0, p0, jnp.where(lane == 1, p1, 0))
    posm_ref[...] = jnp.transpose(pm)[0:8, :]

    bl = lax.broadcasted_iota(
        jnp.int32, (1, 2 * _LANES), 1).astype(jnp.float32) * _BR
    acc = jnp.full((1, 2 * _LANES), -1, jnp.int32)
    lane1 = lax.broadcasted_iota(jnp.int32, (1, _LANES), 1)
    for e in range(_E):
        ab_e = jnp.sum(jnp.where(lane1 == e, ab, 0.0))
        acc = acc + (bl >= ab_e).astype(jnp.int32)
    be_ref[...] = acc


def _group_mm_body(be_ref, x_ref, w_ref, o_ref):
    del be_ref
    xb = _unpack_halves(x_ref[...]).astype(jnp.bfloat16)
    acc = jnp.dot(xb, w_ref[0, :, :], preferred_element_type=jnp.float32)
    o_ref[...] = _pack_halves(acc)


def _combine_body(y0_ref, y1_ref, wts_ref, sc_ref, bp_ref, o_ref):
    wts = wts_ref[...]
    acc = jnp.dot(sc_ref[...], bp_ref[...], preferred_element_type=jnp.float32)
    acc = acc + _unpack_halves(y0_ref[...]) * wts[:, 0:1]
    acc = acc + _unpack_halves(y1_ref[...]) * wts[:, 1:2]
    o_ref[...] = acc


def _sc_scatter(x_hbm, pos0_hbm, pos1_hbm, xs_hbm, rows, idx0, idx1,
                sem0, sem1):
    wid = lax.axis_index("s") * 2 + lax.axis_index("c")
    base = wid * _TPW

    def body(j, carry):
        b = base + j * _CS
        pltpu.sync_copy(pos0_hbm.at[pl.ds(b, _CS)], idx0)
        pltpu.sync_copy(pos1_hbm.at[pl.ds(b, _CS)], idx1)
        pltpu.sync_copy(x_hbm.at[pl.ds(b, _CS)], rows)
        c0 = pltpu.async_copy(rows, xs_hbm.at[idx0], sem0)
        c1 = pltpu.async_copy(rows, xs_hbm.at[idx1], sem1)
        c0.wait()
        c1.wait()
        return carry

    lax.fori_loop(0, _TPW // _CS, body, 0)


def _sc_gather(ys_hbm, pos0_hbm, pos1_hbm, y_hbm, rows0, rows1, idx0, idx1,
               sem0, sem1):
    wid = lax.axis_index("s") * 2 + lax.axis_index("c")
    base = wid * _TPW

    def body(j, carry):
        b = base + j * _CG
        pltpu.sync_copy(pos0_hbm.at[pl.ds(b, _CG)], idx0)
        pltpu.sync_copy(pos1_hbm.at[pl.ds(b, _CG)], idx1)
        c0 = pltpu.async_copy(ys_hbm.at[idx0], rows0, sem0)
        c1 = pltpu.async_copy(ys_hbm.at[idx1], rows1, sem1)
        c0.wait()
        c1.wait()
        pltpu.sync_copy(rows0, y_hbm.at[pl.ds(b, _CG)])
        pltpu.sync_copy(rows1, y_hbm.at[pl.ds(_T + b, _CG)])
        return carry

    lax.fori_loop(0, _TPW // _CG, body, 0)


def _sc_mesh():
    return plsc.VectorSubcoreMesh(core_axis_name="c", subcore_axis_name="s")


def _dispatch_scatter(xpk, pos0, pos1):
    return pl.kernel(
        _sc_scatter,
        mesh=_sc_mesh(),
        out_type=jax.ShapeDtypeStruct((_S, _D // 2), jnp.int32),
        scratch_types=[
            pltpu.VMEM((_CS, _D // 2), jnp.int32),
            pltpu.VMEM((_CS,), jnp.int32),
            pltpu.VMEM((_CS,), jnp.int32),
            pltpu.SemaphoreType.DMA,
            pltpu.SemaphoreType.DMA,
        ],
    )(xpk, pos0, pos1)


def _gather_back(ys, pos0, pos1):
    return pl.kernel(
        _sc_gather,
        mesh=_sc_mesh(),
        out_type=jax.ShapeDtypeStruct((2 * _T, _D // 2), jnp.int32),
        scratch_types=[
            pltpu.VMEM((_CG, _D // 2), jnp.int32),
            pltpu.VMEM((_CG, _D // 2), jnp.int32),
            pltpu.VMEM((_CG,), jnp.int32),
            pltpu.VMEM((_CG,), jnp.int32),
            pltpu.SemaphoreType.DMA,
            pltpu.SemaphoreType.DMA,
        ],
    )(ys, pos0, pos1)


def kernel(x, gate_W, gate_b, expert_W, expert_b):
    n_tb = _T // _BT
    gw_pad = jnp.zeros((_D, _LANES), jnp.float32).at[:, :_E].set(gate_W)
    gb_pad = jnp.full((1, _LANES), _NEG, jnp.float32).at[0, :_E].set(gate_b)
    bp = jnp.zeros((_LANES, _D), jnp.float32).at[:_E].set(expert_b)
    w_bf = expert_W.astype(jnp.bfloat16)

    sc, wts, em, rk, cnt, xpk = pl.pallas_call(
        _route_body,
        grid=(n_tb,),
        in_specs=[
            pl.BlockSpec((_BT, _D), lambda i: (i, 0)),
            pl.BlockSpec((_D, _LANES), lambda i: (0, 0)),
            pl.BlockSpec((1, _LANES), lambda i: (0, 0)),
        ],
        out_specs=[
            pl.BlockSpec((_BT, _LANES), lambda i: (i, 0)),
            pl.BlockSpec((_BT, _LANES), lambda i: (i, 0)),
            pl.BlockSpec((_BT, _LANES), lambda i: (i, 0)),
            pl.BlockSpec((_BT, _LANES), lambda i: (i, 0)),
            pl.BlockSpec((1, _LANES), lambda i: (0, 0)),
            pl.BlockSpec((_BT, _D // 2), lambda i: (i, 0)),
        ],
        out_shape=[
            jax.ShapeDtypeStruct((_T, _LANES), jnp.float32),
            jax.ShapeDtypeStruct((_T, _LANES), jnp.float32),
            jax.ShapeDtypeStruct((_T, _LANES), jnp.int32),
            jax.ShapeDtypeStruct((_T, _LANES), jnp.int32),
            jax.ShapeDtypeStruct((1, _LANES), jnp.int32),
            jax.ShapeDtypeStruct((_T, _D // 2), jnp.int32),
        ],
        scratch_shapes=[pltpu.VMEM((1, _LANES), jnp.float32)],
    )(x, gw_pad, gb_pad)

    posm, be = pl.pallas_call(
        _pos_body,
        grid=(n_tb,),
        in_specs=[
            pl.BlockSpec((1, _LANES), lambda i: (0, 0)),
            pl.BlockSpec((_BT, _LANES), lambda i: (i, 0)),
            pl.BlockSpec((_BT, _LANES), lambda i: (i, 0)),
        ],
        out_specs=[
            pl.BlockSpec((8, _BT), lambda i: (0, i)),
            pl.BlockSpec((1, 2 * _LANES), lambda i: (0, 0)),
        ],
        out_shape=[
            jax.ShapeDtypeStruct((8, _T), jnp.int32),
            jax.ShapeDtypeStruct((1, 2 * _LANES), jnp.int32),
        ],
    )(cnt, em, rk)

    pos0 = posm[0]
    pos1 = posm[1]
    xs = _dispatch_scatter(xpk, pos0, pos1)
    grid_spec = pltpu.PrefetchScalarGridSpec(
        num_scalar_prefetch=1,
        grid=(_NBLK,),
        in_specs=[
            pl.BlockSpec((_BR, _D // 2), lambda nb, be_r: (nb, 0)),
            pl.BlockSpec((1, _D, _D), lambda nb, be_r: (be_r[0, nb], 0, 0)),
        ],
        out_specs=pl.BlockSpec((_BR, _D // 2), lambda nb, be_r: (nb, 0)),
    )
    ys = pl.pallas_call(
        _group_mm_body,
        grid_spec=grid_spec,
        out_shape=jax.ShapeDtypeStruct((_S, _D // 2), jnp.int32),
    )(be, xs, w_bf)

    y = _gather_back(ys, pos0, pos1)
    out = pl.pallas_call(
        _combine_body,
        grid=(n_tb,),
        in_specs=[
            pl.BlockSpec((_BT, _D // 2), lambda i: (i, 0)),
            pl.BlockSpec((_BT, _D // 2), lambda i: (i + n_tb, 0)),
            pl.BlockSpec((_BT, _LANES), lambda i: (i, 0)),
            pl.BlockSpec((_BT, _LANES), lambda i: (i, 0)),
            pl.BlockSpec((_LANES, _D), lambda i: (0, 0)),
        ],
        out_specs=pl.BlockSpec((_BT, _D), lambda i: (i, 0)),
        out_shape=jax.ShapeDtypeStruct((_T, _D), jnp.float32),
    )(y, y, wts, sc, bp)
    return out
